# Initial kernel scaffold; baseline (speedup 1.0000x reference)
import jax
import jax.numpy as jnp
from jax.experimental import pallas as pl


def kernel(x, edge_index, adj_norm, adj_t_norm, so_norm, si_norm, W1s, b1s, W1d, b1d, W2s, b2s, W2d, b2d):
    raise NotImplementedError("write your pallas kernel here")


# speedup vs baseline: 1.0635x; 1.0635x over previous
import functools

import jax
import jax.numpy as jnp
from jax.experimental import pallas as pl

ALPHA1 = 0.5
ALPHA2 = 0.5
BETA1 = 0.7

C_ADJ = BETA1 * ALPHA1
C_ADJT = BETA1 * (1.0 - ALPHA1)
C_SO = (1.0 - BETA1) * ALPHA2
C_SI = (1.0 - BETA1) * (1.0 - ALPHA2)

ROW_BLOCK = 128


def _pass1_kernel(adj_ref, adjt_ref, so_ref, si_ref, x_ref,
                  w1st_ref, w1dt_ref, b1s_ref, b1d_ref,
                  w2st_ref, w2dt_ref,
                  hs_ref, hd_ref):
    cs = C_ADJ * adj_ref[...] + C_SO * so_ref[...]
    cd = C_ADJT * adjt_ref[...] + C_SI * si_ref[...]
    ts = jnp.dot(cs, x_ref[...], preferred_element_type=jnp.float32)
    td = jnp.dot(cd, x_ref[...], preferred_element_type=jnp.float32)
    z = (jnp.dot(ts, w1st_ref[...], preferred_element_type=jnp.float32)
         + jnp.dot(td, w1dt_ref[...], preferred_element_type=jnp.float32)
         + (C_ADJ + C_SO) * b1s_ref[...]
         + (C_ADJT + C_SI) * b1d_ref[...])
    h = jnp.maximum(z, 0.0)
    hs_ref[...] = jnp.dot(h, w2st_ref[...], preferred_element_type=jnp.float32)
    hd_ref[...] = jnp.dot(h, w2dt_ref[...], preferred_element_type=jnp.float32)


def _pass2_kernel(adj_ref, adjt_ref, so_ref, si_ref, hs_ref, hd_ref,
                  b2s_ref, b2d_ref, out_ref):
    cs = C_ADJ * adj_ref[...] + C_SO * so_ref[...]
    cd = C_ADJT * adjt_ref[...] + C_SI * si_ref[...]
    z = (jnp.dot(cs, hs_ref[...], preferred_element_type=jnp.float32)
         + jnp.dot(cd, hd_ref[...], preferred_element_type=jnp.float32)
         + (C_ADJ + C_SO) * b2s_ref[...]
         + (C_ADJT + C_SI) * b2d_ref[...])
    m = jnp.max(z, axis=1, keepdims=True)
    e = jnp.exp(z - m)
    lse = jnp.log(jnp.sum(e, axis=1, keepdims=True))
    out_ref[...] = z - m - lse


def kernel(x, edge_index, adj_norm, adj_t_norm, so_norm, si_norm,
           W1s, b1s, W1d, b1d, W2s, b2s, W2d, b2d):
    del edge_index
    n, d_feat = x.shape
    hidden = W1s.shape[0]
    n_classes = W2s.shape[0]
    rb = ROW_BLOCK
    n_blocks = pl.cdiv(n, rb)

    mat_spec = pl.BlockSpec((rb, n), lambda i: (i, 0))

    def const_spec(shape):
        return pl.BlockSpec(shape, lambda i: (0,) * len(shape))

    hs, hd = pl.pallas_call(
        _pass1_kernel,
        grid=(n_blocks,),
        in_specs=[
            mat_spec, mat_spec, mat_spec, mat_spec,
            const_spec((n, d_feat)),
            const_spec((d_feat, hidden)), const_spec((d_feat, hidden)),
            const_spec((1, hidden)), const_spec((1, hidden)),
            const_spec((hidden, n_classes)), const_spec((hidden, n_classes)),
        ],
        out_specs=[
            pl.BlockSpec((rb, n_classes), lambda i: (i, 0)),
            pl.BlockSpec((rb, n_classes), lambda i: (i, 0)),
        ],
        out_shape=[
            jax.ShapeDtypeStruct((n, n_classes), jnp.float32),
            jax.ShapeDtypeStruct((n, n_classes), jnp.float32),
        ],
    )(adj_norm, adj_t_norm, so_norm, si_norm, x,
      W1s.T, W1d.T, b1s.reshape(1, hidden), b1d.reshape(1, hidden),
      W2s.T, W2d.T)

    out = pl.pallas_call(
        _pass2_kernel,
        grid=(n_blocks,),
        in_specs=[
            mat_spec, mat_spec, mat_spec, mat_spec,
            const_spec((n, n_classes)), const_spec((n, n_classes)),
            const_spec((1, n_classes)), const_spec((1, n_classes)),
        ],
        out_specs=pl.BlockSpec((rb, n_classes), lambda i: (i, 0)),
        out_shape=jax.ShapeDtypeStruct((n, n_classes), jnp.float32),
    )(adj_norm, adj_t_norm, so_norm, si_norm, hs, hd,
      b2s.reshape(1, n_classes), b2d.reshape(1, n_classes))

    return out


# speedup vs baseline: 1.3013x; 1.2236x over previous
import functools

import jax
import jax.numpy as jnp
from jax.experimental import pallas as pl

ALPHA1 = 0.5
ALPHA2 = 0.5
BETA1 = 0.7

C_ADJ = BETA1 * ALPHA1
C_ADJT = BETA1 * (1.0 - ALPHA1)
C_SO = (1.0 - BETA1) * ALPHA2
C_SI = (1.0 - BETA1) * (1.0 - ALPHA2)

ROW_BLOCK = 128


def _pass1_kernel(adj_ref, adjt_ref, so_ref, si_ref, x_ref,
                  w1st_ref, w1dt_ref, b1s_ref, b1d_ref,
                  w2st_ref, w2dt_ref,
                  hs_ref, hd_ref, cs_ref, cd_ref):
    cs = C_ADJ * adj_ref[...] + C_SO * so_ref[...]
    cd = C_ADJT * adjt_ref[...] + C_SI * si_ref[...]
    cs_ref[...] = cs.astype(jnp.bfloat16)
    cd_ref[...] = cd.astype(jnp.bfloat16)
    ts = jnp.dot(cs, x_ref[...], preferred_element_type=jnp.float32)
    td = jnp.dot(cd, x_ref[...], preferred_element_type=jnp.float32)
    z = (jnp.dot(ts, w1st_ref[...], preferred_element_type=jnp.float32)
         + jnp.dot(td, w1dt_ref[...], preferred_element_type=jnp.float32)
         + (C_ADJ + C_SO) * b1s_ref[...]
         + (C_ADJT + C_SI) * b1d_ref[...])
    h = jnp.maximum(z, 0.0)
    hs_ref[...] = jnp.dot(h, w2st_ref[...], preferred_element_type=jnp.float32)
    hd_ref[...] = jnp.dot(h, w2dt_ref[...], preferred_element_type=jnp.float32)


def _pass2_kernel(cs_ref, cd_ref, hs_ref, hd_ref,
                  b2s_ref, b2d_ref, out_ref):
    cs = cs_ref[...].astype(jnp.float32)
    cd = cd_ref[...].astype(jnp.float32)
    z = (jnp.dot(cs, hs_ref[...], preferred_element_type=jnp.float32)
         + jnp.dot(cd, hd_ref[...], preferred_element_type=jnp.float32)
         + (C_ADJ + C_SO) * b2s_ref[...]
         + (C_ADJT + C_SI) * b2d_ref[...])
    m = jnp.max(z, axis=1, keepdims=True)
    e = jnp.exp(z - m)
    lse = jnp.log(jnp.sum(e, axis=1, keepdims=True))
    out_ref[...] = z - m - lse


def kernel(x, edge_index, adj_norm, adj_t_norm, so_norm, si_norm,
           W1s, b1s, W1d, b1d, W2s, b2s, W2d, b2d):
    del edge_index
    n, d_feat = x.shape
    hidden = W1s.shape[0]
    n_classes = W2s.shape[0]
    rb = ROW_BLOCK
    n_blocks = pl.cdiv(n, rb)

    mat_spec = pl.BlockSpec((rb, n), lambda i: (i, 0))

    def const_spec(shape):
        return pl.BlockSpec(shape, lambda i: (0,) * len(shape))

    hs, hd, cs, cd = pl.pallas_call(
        _pass1_kernel,
        grid=(n_blocks,),
        in_specs=[
            mat_spec, mat_spec, mat_spec, mat_spec,
            const_spec((n, d_feat)),
            const_spec((d_feat, hidden)), const_spec((d_feat, hidden)),
            const_spec((1, hidden)), const_spec((1, hidden)),
            const_spec((hidden, n_classes)), const_spec((hidden, n_classes)),
        ],
        out_specs=[
            pl.BlockSpec((rb, n_classes), lambda i: (i, 0)),
            pl.BlockSpec((rb, n_classes), lambda i: (i, 0)),
            mat_spec, mat_spec,
        ],
        out_shape=[
            jax.ShapeDtypeStruct((n, n_classes), jnp.float32),
            jax.ShapeDtypeStruct((n, n_classes), jnp.float32),
            jax.ShapeDtypeStruct((n, n), jnp.bfloat16),
            jax.ShapeDtypeStruct((n, n), jnp.bfloat16),
        ],
    )(adj_norm, adj_t_norm, so_norm, si_norm, x,
      W1s.T, W1d.T, b1s.reshape(1, hidden), b1d.reshape(1, hidden),
      W2s.T, W2d.T)

    out = pl.pallas_call(
        _pass2_kernel,
        grid=(n_blocks,),
        in_specs=[
            mat_spec, mat_spec,
            const_spec((n, n_classes)), const_spec((n, n_classes)),
            const_spec((1, n_classes)), const_spec((1, n_classes)),
        ],
        out_specs=pl.BlockSpec((rb, n_classes), lambda i: (i, 0)),
        out_shape=jax.ShapeDtypeStruct((n, n_classes), jnp.float32),
    )(cs, cd, hs, hd,
      b2s.reshape(1, n_classes), b2d.reshape(1, n_classes))

    return out


# speedup vs baseline: 1.3101x; 1.0068x over previous
import functools

import jax
import jax.numpy as jnp
from jax.experimental import pallas as pl

ALPHA1 = 0.5
ALPHA2 = 0.5
BETA1 = 0.7

C_ADJ = BETA1 * ALPHA1
C_ADJT = BETA1 * (1.0 - ALPHA1)
C_SO = (1.0 - BETA1) * ALPHA2
C_SI = (1.0 - BETA1) * (1.0 - ALPHA2)

ROW_BLOCK = 128


def _pass1_kernel(adj_ref, adjt_ref, so_ref, si_ref, x_ref,
                  w1st_ref, w1dt_ref, b1_ref,
                  w2st_ref, w2dt_ref,
                  hs_ref, hd_ref, cs_ref, cd_ref):
    cs = adj_ref[...] + (C_SO / C_ADJ) * so_ref[...]
    cd = adjt_ref[...] + (C_SI / C_ADJT) * si_ref[...]
    cs_ref[...] = cs.astype(jnp.bfloat16)
    cd_ref[...] = cd.astype(jnp.bfloat16)
    ts = jnp.dot(cs, x_ref[...], preferred_element_type=jnp.float32)
    td = jnp.dot(cd, x_ref[...], preferred_element_type=jnp.float32)
    z = (jnp.dot(ts, w1st_ref[...], preferred_element_type=jnp.float32)
         + jnp.dot(td, w1dt_ref[...], preferred_element_type=jnp.float32)
         + b1_ref[...])
    h = jnp.maximum(z, 0.0)
    hs_ref[...] = jnp.dot(h, w2st_ref[...],
                          preferred_element_type=jnp.float32).astype(jnp.bfloat16)
    hd_ref[...] = jnp.dot(h, w2dt_ref[...],
                          preferred_element_type=jnp.float32).astype(jnp.bfloat16)


def _pass2_kernel(cs_ref, cd_ref, hs_ref, hd_ref, b2_ref, out_ref):
    z = (jnp.dot(cs_ref[...], hs_ref[...], preferred_element_type=jnp.float32)
         + jnp.dot(cd_ref[...], hd_ref[...], preferred_element_type=jnp.float32)
         + b2_ref[...])
    m = jnp.max(z, axis=1, keepdims=True)
    e = jnp.exp(z - m)
    lse = jnp.log(jnp.sum(e, axis=1, keepdims=True))
    out_ref[...] = z - m - lse


def kernel(x, edge_index, adj_norm, adj_t_norm, so_norm, si_norm,
           W1s, b1s, W1d, b1d, W2s, b2s, W2d, b2d):
    del edge_index
    n, d_feat = x.shape
    hidden = W1s.shape[0]
    n_classes = W2s.shape[0]
    rb = ROW_BLOCK
    n_blocks = pl.cdiv(n, rb)

    mat_spec = pl.BlockSpec((rb, n), lambda i: (i, 0))

    def const_spec(shape):
        return pl.BlockSpec(shape, lambda i: (0,) * len(shape))

    w1st = C_ADJ * W1s.T
    w1dt = C_ADJT * W1d.T
    b1 = ((C_ADJ + C_SO) * b1s + (C_ADJT + C_SI) * b1d).reshape(1, hidden)
    w2st = C_ADJ * W2s.T
    w2dt = C_ADJT * W2d.T
    b2 = ((C_ADJ + C_SO) * b2s + (C_ADJT + C_SI) * b2d).reshape(1, n_classes)

    hs, hd, cs, cd = pl.pallas_call(
        _pass1_kernel,
        grid=(n_blocks,),
        in_specs=[
            mat_spec, mat_spec, mat_spec, mat_spec,
            const_spec((n, d_feat)),
            const_spec((d_feat, hidden)), const_spec((d_feat, hidden)),
            const_spec((1, hidden)),
            const_spec((hidden, n_classes)), const_spec((hidden, n_classes)),
        ],
        out_specs=[
            pl.BlockSpec((rb, n_classes), lambda i: (i, 0)),
            pl.BlockSpec((rb, n_classes), lambda i: (i, 0)),
            mat_spec, mat_spec,
        ],
        out_shape=[
            jax.ShapeDtypeStruct((n, n_classes), jnp.bfloat16),
            jax.ShapeDtypeStruct((n, n_classes), jnp.bfloat16),
            jax.ShapeDtypeStruct((n, n), jnp.bfloat16),
            jax.ShapeDtypeStruct((n, n), jnp.bfloat16),
        ],
    )(adj_norm, adj_t_norm, so_norm, si_norm, x,
      w1st, w1dt, b1, w2st, w2dt)

    out = pl.pallas_call(
        _pass2_kernel,
        grid=(n_blocks,),
        in_specs=[
            mat_spec, mat_spec,
            const_spec((n, n_classes)), const_spec((n, n_classes)),
            const_spec((1, n_classes)),
        ],
        out_specs=pl.BlockSpec((rb, n_classes), lambda i: (i, 0)),
        out_shape=jax.ShapeDtypeStruct((n, n_classes), jnp.float32),
    )(cs, cd, hs, hd, b2)

    return out


# speedup vs baseline: 1.3631x; 1.0404x over previous
import functools

import jax
import jax.numpy as jnp
from jax.experimental import pallas as pl

ALPHA1 = 0.5
ALPHA2 = 0.5
BETA1 = 0.7

C_ADJ = BETA1 * ALPHA1
C_ADJT = BETA1 * (1.0 - ALPHA1)
C_SO = (1.0 - BETA1) * ALPHA2
C_SI = (1.0 - BETA1) * (1.0 - ALPHA2)

ROW_BLOCK = 128
ROW_BLOCK_P2 = 512


def _pass1_kernel(adj_ref, adjt_ref, so_ref, si_ref, x_ref,
                  w1st_ref, w1dt_ref, b1_ref,
                  w2st_ref, w2dt_ref,
                  hs_ref, hd_ref, cs_ref, cd_ref):
    cs = adj_ref[...] + (C_SO / C_ADJ) * so_ref[...]
    cd = adjt_ref[...] + (C_SI / C_ADJT) * si_ref[...]
    cs_ref[...] = cs.astype(jnp.bfloat16)
    cd_ref[...] = cd.astype(jnp.bfloat16)
    ts = jnp.dot(cs, x_ref[...], preferred_element_type=jnp.float32)
    td = jnp.dot(cd, x_ref[...], preferred_element_type=jnp.float32)
    z = (jnp.dot(ts, w1st_ref[...], preferred_element_type=jnp.float32)
         + jnp.dot(td, w1dt_ref[...], preferred_element_type=jnp.float32)
         + b1_ref[...])
    h = jnp.maximum(z, 0.0)
    hs_ref[...] = jnp.dot(h, w2st_ref[...],
                          preferred_element_type=jnp.float32).astype(jnp.bfloat16)
    hd_ref[...] = jnp.dot(h, w2dt_ref[...],
                          preferred_element_type=jnp.float32).astype(jnp.bfloat16)


def _pass2_kernel(cs_ref, cd_ref, hs_ref, hd_ref, b2_ref, out_ref):
    z = (jnp.dot(cs_ref[...], hs_ref[...], preferred_element_type=jnp.float32)
         + jnp.dot(cd_ref[...], hd_ref[...], preferred_element_type=jnp.float32)
         + b2_ref[...])
    m = jnp.max(z, axis=1, keepdims=True)
    e = jnp.exp(z - m)
    lse = jnp.log(jnp.sum(e, axis=1, keepdims=True))
    out_ref[...] = z - m - lse


def kernel(x, edge_index, adj_norm, adj_t_norm, so_norm, si_norm,
           W1s, b1s, W1d, b1d, W2s, b2s, W2d, b2d):
    del edge_index
    n, d_feat = x.shape
    hidden = W1s.shape[0]
    n_classes = W2s.shape[0]
    rb = ROW_BLOCK
    n_blocks = pl.cdiv(n, rb)

    mat_spec = pl.BlockSpec((rb, n), lambda i: (i, 0))

    def const_spec(shape):
        return pl.BlockSpec(shape, lambda i: (0,) * len(shape))

    w1st = C_ADJ * W1s.T
    w1dt = C_ADJT * W1d.T
    b1 = ((C_ADJ + C_SO) * b1s + (C_ADJT + C_SI) * b1d).reshape(1, hidden)
    w2st = C_ADJ * W2s.T
    w2dt = C_ADJT * W2d.T
    b2 = ((C_ADJ + C_SO) * b2s + (C_ADJT + C_SI) * b2d).reshape(1, n_classes)

    hs, hd, cs, cd = pl.pallas_call(
        _pass1_kernel,
        grid=(n_blocks,),
        in_specs=[
            mat_spec, mat_spec, mat_spec, mat_spec,
            const_spec((n, d_feat)),
            const_spec((d_feat, hidden)), const_spec((d_feat, hidden)),
            const_spec((1, hidden)),
            const_spec((hidden, n_classes)), const_spec((hidden, n_classes)),
        ],
        out_specs=[
            pl.BlockSpec((rb, n_classes), lambda i: (i, 0)),
            pl.BlockSpec((rb, n_classes), lambda i: (i, 0)),
            mat_spec, mat_spec,
        ],
        out_shape=[
            jax.ShapeDtypeStruct((n, n_classes), jnp.bfloat16),
            jax.ShapeDtypeStruct((n, n_classes), jnp.bfloat16),
            jax.ShapeDtypeStruct((n, n), jnp.bfloat16),
            jax.ShapeDtypeStruct((n, n), jnp.bfloat16),
        ],
    )(adj_norm, adj_t_norm, so_norm, si_norm, x,
      w1st, w1dt, b1, w2st, w2dt)

    rb2 = ROW_BLOCK_P2
    mat_spec2 = pl.BlockSpec((rb2, n), lambda i: (i, 0))
    out = pl.pallas_call(
        _pass2_kernel,
        grid=(pl.cdiv(n, rb2),),
        in_specs=[
            mat_spec2, mat_spec2,
            const_spec((n, n_classes)), const_spec((n, n_classes)),
            const_spec((1, n_classes)),
        ],
        out_specs=pl.BlockSpec((rb2, n_classes), lambda i: (i, 0)),
        out_shape=jax.ShapeDtypeStruct((n, n_classes), jnp.float32),
    )(cs, cd, hs, hd, b2)

    return out


# speedup vs baseline: 1.3648x; 1.0013x over previous
import functools

import jax
import jax.numpy as jnp
from jax.experimental import pallas as pl

ALPHA1 = 0.5
ALPHA2 = 0.5
BETA1 = 0.7

C_ADJ = BETA1 * ALPHA1
C_ADJT = BETA1 * (1.0 - ALPHA1)
C_SO = (1.0 - BETA1) * ALPHA2
C_SI = (1.0 - BETA1) * (1.0 - ALPHA2)

ROW_BLOCK = 128
ROW_BLOCK_P2 = 576


def _pass1_kernel(adj_ref, adjt_ref, so_ref, si_ref, x_ref,
                  w1st_ref, w1dt_ref, b1_ref,
                  w2st_ref, w2dt_ref,
                  hs_ref, hd_ref, cs_ref, cd_ref):
    cs = adj_ref[...] + (C_SO / C_ADJ) * so_ref[...]
    cd = adjt_ref[...] + (C_SI / C_ADJT) * si_ref[...]
    cs_ref[...] = cs.astype(jnp.bfloat16)
    cd_ref[...] = cd.astype(jnp.bfloat16)
    ts = jnp.dot(cs, x_ref[...], preferred_element_type=jnp.float32)
    td = jnp.dot(cd, x_ref[...], preferred_element_type=jnp.float32)
    z = (jnp.dot(ts, w1st_ref[...], preferred_element_type=jnp.float32)
         + jnp.dot(td, w1dt_ref[...], preferred_element_type=jnp.float32)
         + b1_ref[...])
    h = jnp.maximum(z, 0.0)
    hs_ref[...] = jnp.dot(h, w2st_ref[...],
                          preferred_element_type=jnp.float32).astype(jnp.bfloat16)
    hd_ref[...] = jnp.dot(h, w2dt_ref[...],
                          preferred_element_type=jnp.float32).astype(jnp.bfloat16)


def _pass2_kernel(cs_ref, cd_ref, hs_ref, hd_ref, b2_ref, out_ref):
    z = (jnp.dot(cs_ref[...], hs_ref[...], preferred_element_type=jnp.float32)
         + jnp.dot(cd_ref[...], hd_ref[...], preferred_element_type=jnp.float32)
         + b2_ref[...])
    m = jnp.max(z, axis=1, keepdims=True)
    e = jnp.exp(z - m)
    lse = jnp.log(jnp.sum(e, axis=1, keepdims=True))
    out_ref[...] = z - m - lse


def kernel(x, edge_index, adj_norm, adj_t_norm, so_norm, si_norm,
           W1s, b1s, W1d, b1d, W2s, b2s, W2d, b2d):
    del edge_index
    n, d_feat = x.shape
    hidden = W1s.shape[0]
    n_classes = W2s.shape[0]
    rb = ROW_BLOCK
    n_blocks = pl.cdiv(n, rb)

    mat_spec = pl.BlockSpec((rb, n), lambda i: (i, 0))

    def const_spec(shape):
        return pl.BlockSpec(shape, lambda i: (0,) * len(shape))

    w1st = C_ADJ * W1s.T
    w1dt = C_ADJT * W1d.T
    b1 = ((C_ADJ + C_SO) * b1s + (C_ADJT + C_SI) * b1d).reshape(1, hidden)
    w2st = C_ADJ * W2s.T
    w2dt = C_ADJT * W2d.T
    b2 = ((C_ADJ + C_SO) * b2s + (C_ADJT + C_SI) * b2d).reshape(1, n_classes)

    hs, hd, cs, cd = pl.pallas_call(
        _pass1_kernel,
        grid=(n_blocks,),
        in_specs=[
            mat_spec, mat_spec, mat_spec, mat_spec,
            const_spec((n, d_feat)),
            const_spec((d_feat, hidden)), const_spec((d_feat, hidden)),
            const_spec((1, hidden)),
            const_spec((hidden, n_classes)), const_spec((hidden, n_classes)),
        ],
        out_specs=[
            pl.BlockSpec((rb, n_classes), lambda i: (i, 0)),
            pl.BlockSpec((rb, n_classes), lambda i: (i, 0)),
            mat_spec, mat_spec,
        ],
        out_shape=[
            jax.ShapeDtypeStruct((n, n_classes), jnp.bfloat16),
            jax.ShapeDtypeStruct((n, n_classes), jnp.bfloat16),
            jax.ShapeDtypeStruct((n, n), jnp.bfloat16),
            jax.ShapeDtypeStruct((n, n), jnp.bfloat16),
        ],
    )(adj_norm, adj_t_norm, so_norm, si_norm, x,
      w1st, w1dt, b1, w2st, w2dt)

    rb2 = ROW_BLOCK_P2
    mat_spec2 = pl.BlockSpec((rb2, n), lambda i: (i, 0))
    out = pl.pallas_call(
        _pass2_kernel,
        grid=(pl.cdiv(n, rb2),),
        in_specs=[
            mat_spec2, mat_spec2,
            const_spec((n, n_classes)), const_spec((n, n_classes)),
            const_spec((1, n_classes)),
        ],
        out_specs=pl.BlockSpec((rb2, n_classes), lambda i: (i, 0)),
        out_shape=jax.ShapeDtypeStruct((n, n_classes), jnp.float32),
    )(cs, cd, hs, hd, b2)

    return out
